# baseline (speedup 1.0000x reference)
import functools

import jax
import jax.numpy as jnp
from jax import lax
from jax.experimental import pallas as pl
from jax.experimental.pallas import tpu as pltpu
from jax.experimental.pallas import tpu_sc as plsc

B, T, D, H, E, TOPK = 256, 8, 1024, 2048, 8, 2
N = B * T
TILE = 256
NT = N // TILE
TE = T * E
NEG = -1e30
BM = 128
A = N * TOPK
NTILES = A // BM + E
P = NTILES * BM
NW = 32
TPW = N // NW
CHUNK = 16
NCH = TPW // CHUNK


def _routing_body(x_ref, wgt_ref,
                  pos0_ref, pos1_ref, w0_ref, w1_ref, gid_ref, aux_ref,
                  e1s, e2s, r0s, r1s, w0s, w1s, rb, ps):
    i = pl.program_id(0)

    @pl.when(i == 0)
    def _():
        rb[...] = jnp.zeros((1, E), jnp.int32)
        ps[...] = jnp.zeros((1, TE), jnp.float32)

    @pl.when(i < NT)
    def _():
        xt = x_ref[...]
        logits = jnp.dot(xt, wgt_ref[...],
                         preferred_element_type=jnp.float32)
        rows = jax.lax.broadcasted_iota(jnp.int32, (TILE, TE), 0)
        cols = jax.lax.broadcasted_iota(jnp.int32, (TILE, TE), 1)
        valid = (cols // E) == (rows % T)
        ml = jnp.where(valid, logits, NEG)
        m1 = jnp.max(ml, axis=1, keepdims=True)
        i1 = jnp.min(jnp.where(ml >= m1, cols, TE), axis=1, keepdims=True)
        ml2 = jnp.where(cols == i1, NEG, ml)
        m2 = jnp.max(ml2, axis=1, keepdims=True)
        i2 = jnp.min(jnp.where(ml2 >= m2, cols, TE), axis=1, keepdims=True)
        w1 = 1.0 / (1.0 + jnp.exp(m2 - m1))
        w2 = 1.0 - w1
        e1 = i1 % E
        e2 = i2 % E
        e8 = jax.lax.broadcasted_iota(jnp.int32, (TILE, E), 1)
        oh1 = (e8 == e1)
        oh2 = (e8 == e2)
        mtot = oh1.astype(jnp.float32) + oh2.astype(jnp.float32)
        rr = jax.lax.broadcasted_iota(jnp.int32, (TILE, TILE), 0)
        cc = jax.lax.broadcasted_iota(jnp.int32, (TILE, TILE), 1)
        ltri = (rr > cc).astype(jnp.float32)
        cex = jnp.dot(ltri, mtot,
                      preferred_element_type=jnp.float32).astype(jnp.int32)
        rbb = jnp.broadcast_to(rb[...], (TILE, E))
        rank0 = jnp.sum(jnp.where(oh1, cex + rbb, 0), axis=1, keepdims=True)
        rank1 = jnp.sum(jnp.where(oh2, cex + rbb, 0), axis=1, keepdims=True)
        colsel = (jax.lax.broadcasted_iota(jnp.int32, (TILE, NT), 1) == i)
        e1s[...] = jnp.where(colsel, e1, e1s[...])
        e2s[...] = jnp.where(colsel, e2, e2s[...])
        r0s[...] = jnp.where(colsel, rank0, r0s[...])
        r1s[...] = jnp.where(colsel, rank1, r1s[...])
        rb[...] = rb[...] + jnp.sum(mtot, axis=0, keepdims=True
                                    ).astype(jnp.int32)
        p = jnp.exp(ml - m1)
        p = p / jnp.sum(p, axis=1, keepdims=True)
        ps[...] = ps[...] + jnp.sum(p, axis=0, keepdims=True)
        w0s[...] = jnp.where(colsel, w1, w0s[...])
        w1s[...] = jnp.where(colsel, w2, w1s[...])

    @pl.when(i == NT)
    def _():
        cnt = rb[...]
        padcnt = ((cnt + (BM - 1)) // BM) * BM
        po = []
        run = jnp.int32(0)
        for e in range(E):
            po.append(run)
            run = run + padcnt[0, e]
        sel0 = jnp.zeros((TILE, NT), jnp.int32)
        sel1 = jnp.zeros((TILE, NT), jnp.int32)
        for e in range(E):
            sel0 = sel0 + jnp.where(e1s[...] == e, po[e], 0)
            sel1 = sel1 + jnp.where(e2s[...] == e, po[e], 0)
        pos0_ref[...] = jnp.transpose(r0s[...] + sel0, (1, 0))
        pos1_ref[...] = jnp.transpose(r1s[...] + sel1, (1, 0))
        oh = (jax.lax.broadcasted_iota(jnp.int32, (N, NT), 0) // TILE
              == jax.lax.broadcasted_iota(jnp.int32, (N, NT), 1)
              ).astype(jnp.float32)
        ones16 = jnp.ones((NT, 16), jnp.float32)

        def _tok16(ws):
            m = jnp.concatenate([ws] * NT, axis=0) * oh
            return jnp.dot(m, ones16, preferred_element_type=jnp.float32,
                           precision=jax.lax.Precision.HIGHEST)

        w0_ref[...] = _tok16(w0s[...])
        w1_ref[...] = _tok16(w1s[...])
        tcol = jax.lax.broadcasted_iota(jnp.int32, (1, TE), 1) * BM
        g = jnp.zeros((1, TE), jnp.int32)
        for e in range(E):
            g = g + (tcol >= (po[e] + padcnt[0, e])).astype(jnp.int32)
        gid_ref[...] = jnp.minimum(g, E - 1)
        j64 = jax.lax.broadcasted_iota(jnp.int32, (TE, E), 0)
        k8 = jax.lax.broadcasted_iota(jnp.int32, (TE, E), 1)
        fold = ((j64 % E) == k8).astype(jnp.float32)
        avg = jnp.dot(ps[...], fold, preferred_element_type=jnp.float32,
                      precision=jax.lax.Precision.HIGHEST) / float(N)
        aux_ref[...] = jnp.broadcast_to(float(E) * jnp.sum(avg * avg),
                                        (1, TE))


def _routing(x2, wgt):
    return pl.pallas_call(
        _routing_body,
        grid=(NT + 1,),
        in_specs=[
            pl.BlockSpec((TILE, D), lambda i: (jnp.minimum(i, NT - 1), 0)),
            pl.BlockSpec((D, TE), lambda i: (0, 0)),
        ],
        out_specs=[
            pl.BlockSpec((NT, TILE), lambda i: (0, 0)),
            pl.BlockSpec((NT, TILE), lambda i: (0, 0)),
            pl.BlockSpec((N, 16), lambda i: (0, 0)),
            pl.BlockSpec((N, 16), lambda i: (0, 0)),
            pl.BlockSpec((1, TE), lambda i: (0, 0)),
            pl.BlockSpec((1, TE), lambda i: (0, 0)),
        ],
        out_shape=[
            jax.ShapeDtypeStruct((NT, TILE), jnp.int32),
            jax.ShapeDtypeStruct((NT, TILE), jnp.int32),
            jax.ShapeDtypeStruct((N, 16), jnp.float32),
            jax.ShapeDtypeStruct((N, 16), jnp.float32),
            jax.ShapeDtypeStruct((1, TE), jnp.int32),
            jax.ShapeDtypeStruct((1, TE), jnp.float32),
        ],
        scratch_shapes=[
            pltpu.VMEM((TILE, NT), jnp.int32),
            pltpu.VMEM((TILE, NT), jnp.int32),
            pltpu.VMEM((TILE, NT), jnp.int32),
            pltpu.VMEM((TILE, NT), jnp.int32),
            pltpu.VMEM((TILE, NT), jnp.float32),
            pltpu.VMEM((TILE, NT), jnp.float32),
            pltpu.VMEM((1, E), jnp.int32),
            pltpu.VMEM((1, TE), jnp.float32),
        ],
        compiler_params=pltpu.CompilerParams(
            dimension_semantics=("arbitrary",)),
    )(x2, wgt)


NSPLIT = 4
HS = H // NSPLIT


def _mlp_body(gid_ref, xs_ref, *refs):
    w1s = refs[:NSPLIT]
    w2s = refs[NSPLIT:2 * NSPLIT]
    b1_ref = refs[2 * NSPLIT]
    b2_ref = refs[2 * NSPLIT + 1]
    ys_ref = refs[2 * NSPLIT + 2]
    x = xs_ref[...]
    acc = jnp.broadcast_to(b2_ref[0], (BM, D))
    for j in range(NSPLIT):
        h = jnp.maximum(
            jnp.dot(x, w1s[j][0], preferred_element_type=jnp.float32)
            + b1_ref[0, :, j * HS:(j + 1) * HS], 0.0)
        acc = acc + jnp.dot(h, w2s[j][0], preferred_element_type=jnp.float32)
    ys_ref[...] = acc


def _grouped_mlp(gid, xs, W1, W2, b1r, b2r):
    w1_specs = [
        pl.BlockSpec((1, D, HS), lambda g, s, j=j: (s[g], 0, j))
        for j in range(NSPLIT)
    ]
    w2_specs = [
        pl.BlockSpec((1, HS, D), lambda g, s, j=j: (s[g], j, 0))
        for j in range(NSPLIT)
    ]
    grid_spec = pltpu.PrefetchScalarGridSpec(
        num_scalar_prefetch=1,
        grid=(NTILES,),
        in_specs=[
            pl.BlockSpec((BM, D), lambda g, s: (g, 0)),
            *w1_specs,
            *w2_specs,
            pl.BlockSpec((1, 1, H), lambda g, s: (s[g], 0, 0)),
            pl.BlockSpec((1, 1, D), lambda g, s: (s[g], 0, 0)),
        ],
        out_specs=pl.BlockSpec((BM, D), lambda g, s: (g, 0)),
    )
    return pl.pallas_call(
        _mlp_body,
        grid_spec=grid_spec,
        out_shape=jax.ShapeDtypeStruct((P, D), jnp.float32),
        compiler_params=pltpu.CompilerParams(
            dimension_semantics=("arbitrary",)),
    )(gid, xs, *([W1] * NSPLIT), *([W2] * NSPLIT), b1r, b2r)


@functools.lru_cache(maxsize=None)
def _sc_kernels():
    mesh = plsc.VectorSubcoreMesh(core_axis_name="c", subcore_axis_name="s")
    dispatch = pl.kernel(
        _dispatch_body,
        mesh=mesh,
        out_type=jax.ShapeDtypeStruct((P, D), jnp.float32),
        scratch_types=[
            pltpu.VMEM((TPW, D), jnp.float32),
            pltpu.VMEM((TPW,), jnp.int32),
            pltpu.VMEM((TPW,), jnp.int32),
            pltpu.SemaphoreType.DMA,
            pltpu.SemaphoreType.DMA,
        ],
    )
    combine = pl.kernel(
        _combine_body,
        mesh=mesh,
        out_type=jax.ShapeDtypeStruct((N, D), jnp.float32),
        scratch_types=[
            pltpu.VMEM((2 * CHUNK, D), jnp.float32),
            pltpu.VMEM((2 * CHUNK, D), jnp.float32),
            pltpu.VMEM((2 * CHUNK, D), jnp.float32),
            pltpu.VMEM((2, CHUNK), jnp.int32),
            pltpu.VMEM((2, CHUNK), jnp.int32),
            pltpu.VMEM((2, CHUNK, 16), jnp.float32),
            pltpu.VMEM((2, CHUNK, 16), jnp.float32),
            pltpu.SemaphoreType.DMA,
            pltpu.SemaphoreType.DMA,
            pltpu.SemaphoreType.DMA,
            pltpu.SemaphoreType.DMA,
            pltpu.SemaphoreType.DMA,
            pltpu.SemaphoreType.DMA,
        ],
    )
    return dispatch, combine


def _dispatch_body(x_hbm, p0_hbm, p1_hbm, xs_hbm, xbuf, i0, i1, sem, sem2):
    wid = lax.axis_index("s") * 2 + lax.axis_index("c")
    tb = wid * TPW
    cx = pltpu.async_copy(x_hbm.at[pl.ds(tb, TPW)], xbuf, sem)
    pltpu.sync_copy(p0_hbm.at[pl.ds(tb, TPW)], i0)
    pltpu.sync_copy(p1_hbm.at[pl.ds(tb, TPW)], i1)
    cx.wait()
    c0 = pltpu.async_copy(xbuf, xs_hbm.at[i0], sem)
    c1 = pltpu.async_copy(xbuf, xs_hbm.at[i1], sem2)
    c0.wait()
    c1.wait()


def _combine_body(ys_hbm, p0_hbm, p1_hbm, w0_hbm, w1_hbm, out_hbm,
                  y0, y1, ob, i0, i1, wb0, wb1,
                  sg0a, sg0b, sg1a, sg1b, sta, stb):
    wid = lax.axis_index("s") * 2 + lax.axis_index("c")
    sg0 = (sg0a, sg0b)
    sg1 = (sg1a, sg1b)
    st = (sta, stb)

    def issue(ch):
        b = ch % 2
        base = wid * TPW + ch * CHUNK
        rows = pl.ds(b * CHUNK, CHUNK)
        pltpu.sync_copy(p0_hbm.at[pl.ds(base, CHUNK)], i0.at[b])
        pltpu.sync_copy(p1_hbm.at[pl.ds(base, CHUNK)], i1.at[b])
        pltpu.sync_copy(w0_hbm.at[pl.ds(base, CHUNK)], wb0.at[b])
        pltpu.sync_copy(w1_hbm.at[pl.ds(base, CHUNK)], wb1.at[b])
        c0 = pltpu.async_copy(ys_hbm.at[i0.at[b]], y0.at[rows], sg0[b])
        c1 = pltpu.async_copy(ys_hbm.at[i1.at[b]], y1.at[rows], sg1[b])
        return c0, c1

    pend = issue(0)
    stp = [None, None]
    for ch in range(NCH):
        b = ch % 2
        cur = pend
        if ch + 1 < NCH:
            pend = issue(ch + 1)
        cur[0].wait()
        cur[1].wait()
        if stp[b] is not None:
            stp[b].wait()
        for r in range(CHUNK):
            row = b * CHUNK + r
            w0r = wb0[b, r]
            w1r = wb1[b, r]

            @plsc.parallel_loop(0, D // 16, 1, unroll=8)
            def _(c):
                sl = pl.ds(c * 16, 16)
                ob[row, sl] = w0r * y0[row, sl] + w1r * y1[row, sl]

        base = wid * TPW + ch * CHUNK
        stp[b] = pltpu.async_copy(ob.at[pl.ds(b * CHUNK, CHUNK)],
                                  out_hbm.at[pl.ds(base, CHUNK)], st[b])
    for s in stp:
        if s is not None:
            s.wait()


@jax.jit
def kernel(x, Wg, W1, b1, W2, b2):
    x2 = x.reshape(N, D)
    wgt = Wg.reshape(TE, D).T
    pos0, pos1, w0v, w1v, gid64, aux = _routing(x2, wgt)
    pos0f = pos0.reshape(N)
    pos1f = pos1.reshape(N)
    gid = gid64[0, :NTILES]
    dispatch, combine = _sc_kernels()
    xs = dispatch(x2, pos0f, pos1f)
    ys = _grouped_mlp(gid, xs, W1, W2,
                      b1.reshape(E, 1, H), b2.reshape(E, 1, D))
    out2 = combine(ys, pos0f, pos1f, w0v, w1v)
    return out2.reshape(B, T, D), aux[0, 0]


# speedup vs baseline: 1.1051x; 1.1051x over previous
import functools

import jax
import jax.numpy as jnp
from jax import lax
from jax.experimental import pallas as pl
from jax.experimental.pallas import tpu as pltpu
from jax.experimental.pallas import tpu_sc as plsc

B, T, D, H, E, TOPK = 256, 8, 1024, 2048, 8, 2
N = B * T
TILE = 256
NT = N // TILE
TE = T * E
NEG = -1e30
BM = 128
A = N * TOPK
NTILES = A // BM + E
P = NTILES * BM
NW = 32
TPW = N // NW
CHUNK = 16
NCH = TPW // CHUNK


def _routing_body(x_ref, wgt_ref,
                  pos0_ref, pos1_ref, w0_ref, w1_ref, gid_ref, aux_ref,
                  e1s, e2s, r0s, r1s, w0s, w1s, rb, ps):
    i = pl.program_id(0)

    @pl.when(i == 0)
    def _():
        rb[...] = jnp.zeros((1, E), jnp.int32)
        ps[...] = jnp.zeros((1, TE), jnp.float32)

    @pl.when(i < NT)
    def _():
        xt = x_ref[...]
        logits = jnp.dot(xt, wgt_ref[...],
                         preferred_element_type=jnp.float32)
        rows = jax.lax.broadcasted_iota(jnp.int32, (TILE, TE), 0)
        cols = jax.lax.broadcasted_iota(jnp.int32, (TILE, TE), 1)
        valid = (cols // E) == (rows % T)
        ml = jnp.where(valid, logits, NEG)
        m1 = jnp.max(ml, axis=1, keepdims=True)
        i1 = jnp.min(jnp.where(ml >= m1, cols, TE), axis=1, keepdims=True)
        ml2 = jnp.where(cols == i1, NEG, ml)
        m2 = jnp.max(ml2, axis=1, keepdims=True)
        i2 = jnp.min(jnp.where(ml2 >= m2, cols, TE), axis=1, keepdims=True)
        w1 = 1.0 / (1.0 + jnp.exp(m2 - m1))
        w2 = 1.0 - w1
        e1 = i1 % E
        e2 = i2 % E
        e8 = jax.lax.broadcasted_iota(jnp.int32, (TILE, E), 1)
        oh1 = (e8 == e1)
        oh2 = (e8 == e2)
        mtot = oh1.astype(jnp.float32) + oh2.astype(jnp.float32)
        rr = jax.lax.broadcasted_iota(jnp.int32, (TILE, TILE), 0)
        cc = jax.lax.broadcasted_iota(jnp.int32, (TILE, TILE), 1)
        ltri = (rr > cc).astype(jnp.float32)
        cex = jnp.dot(ltri, mtot,
                      preferred_element_type=jnp.float32).astype(jnp.int32)
        rbb = jnp.broadcast_to(rb[...], (TILE, E))
        rank0 = jnp.sum(jnp.where(oh1, cex + rbb, 0), axis=1, keepdims=True)
        rank1 = jnp.sum(jnp.where(oh2, cex + rbb, 0), axis=1, keepdims=True)
        colsel = (jax.lax.broadcasted_iota(jnp.int32, (TILE, NT), 1) == i)
        e1s[...] = jnp.where(colsel, e1, e1s[...])
        e2s[...] = jnp.where(colsel, e2, e2s[...])
        r0s[...] = jnp.where(colsel, rank0, r0s[...])
        r1s[...] = jnp.where(colsel, rank1, r1s[...])
        rb[...] = rb[...] + jnp.sum(mtot, axis=0, keepdims=True
                                    ).astype(jnp.int32)
        p = jnp.exp(ml - m1)
        p = p / jnp.sum(p, axis=1, keepdims=True)
        ps[...] = ps[...] + jnp.sum(p, axis=0, keepdims=True)
        w0s[...] = jnp.where(colsel, w1, w0s[...])
        w1s[...] = jnp.where(colsel, w2, w1s[...])

    @pl.when(i == NT)
    def _():
        cnt = rb[...]
        padcnt = ((cnt + (BM - 1)) // BM) * BM
        po = []
        run = jnp.int32(0)
        for e in range(E):
            po.append(run)
            run = run + padcnt[0, e]
        sel0 = jnp.zeros((TILE, NT), jnp.int32)
        sel1 = jnp.zeros((TILE, NT), jnp.int32)
        for e in range(E):
            sel0 = sel0 + jnp.where(e1s[...] == e, po[e], 0)
            sel1 = sel1 + jnp.where(e2s[...] == e, po[e], 0)
        pos0_ref[...] = jnp.transpose(r0s[...] + sel0, (1, 0))
        pos1_ref[...] = jnp.transpose(r1s[...] + sel1, (1, 0))
        oh = (jax.lax.broadcasted_iota(jnp.int32, (N, NT), 0) // TILE
              == jax.lax.broadcasted_iota(jnp.int32, (N, NT), 1)
              ).astype(jnp.float32)
        ones16 = jnp.ones((NT, 16), jnp.float32)

        def _tok16(ws):
            m = jnp.concatenate([ws] * NT, axis=0) * oh
            return jnp.dot(m, ones16, preferred_element_type=jnp.float32,
                           precision=jax.lax.Precision.HIGHEST)

        w0_ref[...] = _tok16(w0s[...])
        w1_ref[...] = _tok16(w1s[...])
        tcol = jax.lax.broadcasted_iota(jnp.int32, (1, TE), 1) * BM
        g = jnp.zeros((1, TE), jnp.int32)
        for e in range(E):
            g = g + (tcol >= (po[e] + padcnt[0, e])).astype(jnp.int32)
        gid_ref[...] = jnp.minimum(g, E - 1)
        j64 = jax.lax.broadcasted_iota(jnp.int32, (TE, E), 0)
        k8 = jax.lax.broadcasted_iota(jnp.int32, (TE, E), 1)
        fold = ((j64 % E) == k8).astype(jnp.float32)
        avg = jnp.dot(ps[...], fold, preferred_element_type=jnp.float32,
                      precision=jax.lax.Precision.HIGHEST) / float(N)
        aux_ref[...] = jnp.broadcast_to(float(E) * jnp.sum(avg * avg),
                                        (1, TE))


def _routing(x2, wgt):
    return pl.pallas_call(
        _routing_body,
        grid=(NT + 1,),
        in_specs=[
            pl.BlockSpec((TILE, D), lambda i: (jnp.minimum(i, NT - 1), 0)),
            pl.BlockSpec((D, TE), lambda i: (0, 0)),
        ],
        out_specs=[
            pl.BlockSpec((NT, TILE), lambda i: (0, 0)),
            pl.BlockSpec((NT, TILE), lambda i: (0, 0)),
            pl.BlockSpec((N, 16), lambda i: (0, 0)),
            pl.BlockSpec((N, 16), lambda i: (0, 0)),
            pl.BlockSpec((1, TE), lambda i: (0, 0)),
            pl.BlockSpec((1, TE), lambda i: (0, 0)),
        ],
        out_shape=[
            jax.ShapeDtypeStruct((NT, TILE), jnp.int32),
            jax.ShapeDtypeStruct((NT, TILE), jnp.int32),
            jax.ShapeDtypeStruct((N, 16), jnp.float32),
            jax.ShapeDtypeStruct((N, 16), jnp.float32),
            jax.ShapeDtypeStruct((1, TE), jnp.int32),
            jax.ShapeDtypeStruct((1, TE), jnp.float32),
        ],
        scratch_shapes=[
            pltpu.VMEM((TILE, NT), jnp.int32),
            pltpu.VMEM((TILE, NT), jnp.int32),
            pltpu.VMEM((TILE, NT), jnp.int32),
            pltpu.VMEM((TILE, NT), jnp.int32),
            pltpu.VMEM((TILE, NT), jnp.float32),
            pltpu.VMEM((TILE, NT), jnp.float32),
            pltpu.VMEM((1, E), jnp.int32),
            pltpu.VMEM((1, TE), jnp.float32),
        ],
        compiler_params=pltpu.CompilerParams(
            dimension_semantics=("arbitrary",)),
    )(x2, wgt)


NSPLIT = 2
HS = H // NSPLIT


def _mlp_body(gid_ref, xs_ref, *refs):
    w1s = refs[:NSPLIT]
    w2s = refs[NSPLIT:2 * NSPLIT]
    b1_ref = refs[2 * NSPLIT]
    b2_ref = refs[2 * NSPLIT + 1]
    ys_ref = refs[2 * NSPLIT + 2]
    x = xs_ref[...]
    acc = jnp.broadcast_to(b2_ref[0], (BM, D))
    for j in range(NSPLIT):
        h = jnp.maximum(
            jnp.dot(x, w1s[j][0], preferred_element_type=jnp.float32)
            + b1_ref[0, :, j * HS:(j + 1) * HS], 0.0)
        acc = acc + jnp.dot(h, w2s[j][0], preferred_element_type=jnp.float32)
    ys_ref[...] = acc


def _grouped_mlp(gid, xs, W1, W2, b1r, b2r):
    w1_specs = [
        pl.BlockSpec((1, D, HS), lambda g, s, j=j: (s[g], 0, j))
        for j in range(NSPLIT)
    ]
    w2_specs = [
        pl.BlockSpec((1, HS, D), lambda g, s, j=j: (s[g], j, 0))
        for j in range(NSPLIT)
    ]
    grid_spec = pltpu.PrefetchScalarGridSpec(
        num_scalar_prefetch=1,
        grid=(NTILES,),
        in_specs=[
            pl.BlockSpec((BM, D), lambda g, s: (g, 0)),
            *w1_specs,
            *w2_specs,
            pl.BlockSpec((1, 1, H), lambda g, s: (s[g], 0, 0)),
            pl.BlockSpec((1, 1, D), lambda g, s: (s[g], 0, 0)),
        ],
        out_specs=pl.BlockSpec((BM, D), lambda g, s: (g, 0)),
    )
    return pl.pallas_call(
        _mlp_body,
        grid_spec=grid_spec,
        out_shape=jax.ShapeDtypeStruct((P, D), jnp.float32),
        compiler_params=pltpu.CompilerParams(
            dimension_semantics=("arbitrary",)),
    )(gid, xs, *([W1] * NSPLIT), *([W2] * NSPLIT), b1r, b2r)


@functools.lru_cache(maxsize=None)
def _sc_kernels():
    mesh = plsc.VectorSubcoreMesh(core_axis_name="c", subcore_axis_name="s")
    dispatch = pl.kernel(
        _dispatch_body,
        mesh=mesh,
        out_type=jax.ShapeDtypeStruct((P, D), jnp.float32),
        scratch_types=[
            pltpu.VMEM((TPW, D), jnp.float32),
            pltpu.VMEM((TPW,), jnp.int32),
            pltpu.VMEM((TPW,), jnp.int32),
            pltpu.SemaphoreType.DMA,
            pltpu.SemaphoreType.DMA,
        ],
    )
    combine = pl.kernel(
        _combine_body,
        mesh=mesh,
        out_type=jax.ShapeDtypeStruct((N, D), jnp.float32),
        scratch_types=[
            pltpu.VMEM((2 * CHUNK, D), jnp.float32),
            pltpu.VMEM((2 * CHUNK, D), jnp.float32),
            pltpu.VMEM((2 * CHUNK, D), jnp.float32),
            pltpu.VMEM((2, CHUNK), jnp.int32),
            pltpu.VMEM((2, CHUNK), jnp.int32),
            pltpu.VMEM((2, CHUNK, 16), jnp.float32),
            pltpu.VMEM((2, CHUNK, 16), jnp.float32),
            pltpu.SemaphoreType.DMA,
            pltpu.SemaphoreType.DMA,
            pltpu.SemaphoreType.DMA,
            pltpu.SemaphoreType.DMA,
            pltpu.SemaphoreType.DMA,
            pltpu.SemaphoreType.DMA,
        ],
    )
    return dispatch, combine


def _dispatch_body(x_hbm, p0_hbm, p1_hbm, xs_hbm, xbuf, i0, i1, sem, sem2):
    wid = lax.axis_index("s") * 2 + lax.axis_index("c")
    tb = wid * TPW
    cx = pltpu.async_copy(x_hbm.at[pl.ds(tb, TPW)], xbuf, sem)
    pltpu.sync_copy(p0_hbm.at[pl.ds(tb, TPW)], i0)
    pltpu.sync_copy(p1_hbm.at[pl.ds(tb, TPW)], i1)
    cx.wait()
    c0 = pltpu.async_copy(xbuf, xs_hbm.at[i0], sem)
    c1 = pltpu.async_copy(xbuf, xs_hbm.at[i1], sem2)
    c0.wait()
    c1.wait()


def _combine_body(ys_hbm, p0_hbm, p1_hbm, w0_hbm, w1_hbm, out_hbm,
                  y0, y1, ob, i0, i1, wb0, wb1,
                  sg0a, sg0b, sg1a, sg1b, sta, stb):
    wid = lax.axis_index("s") * 2 + lax.axis_index("c")
    sg0 = (sg0a, sg0b)
    sg1 = (sg1a, sg1b)
    st = (sta, stb)

    def issue(ch):
        b = ch % 2
        base = wid * TPW + ch * CHUNK
        rows = pl.ds(b * CHUNK, CHUNK)
        pltpu.sync_copy(p0_hbm.at[pl.ds(base, CHUNK)], i0.at[b])
        pltpu.sync_copy(p1_hbm.at[pl.ds(base, CHUNK)], i1.at[b])
        pltpu.sync_copy(w0_hbm.at[pl.ds(base, CHUNK)], wb0.at[b])
        pltpu.sync_copy(w1_hbm.at[pl.ds(base, CHUNK)], wb1.at[b])
        c0 = pltpu.async_copy(ys_hbm.at[i0.at[b]], y0.at[rows], sg0[b])
        c1 = pltpu.async_copy(ys_hbm.at[i1.at[b]], y1.at[rows], sg1[b])
        return c0, c1

    pend = issue(0)
    stp = [None, None]
    for ch in range(NCH):
        b = ch % 2
        cur = pend
        if ch + 1 < NCH:
            pend = issue(ch + 1)
        cur[0].wait()
        cur[1].wait()
        if stp[b] is not None:
            stp[b].wait()
        for r in range(CHUNK):
            row = b * CHUNK + r
            w0r = wb0[b, r]
            w1r = wb1[b, r]

            @plsc.parallel_loop(0, D // 16, 1, unroll=8)
            def _(c):
                sl = pl.ds(c * 16, 16)
                ob[row, sl] = w0r * y0[row, sl] + w1r * y1[row, sl]

        base = wid * TPW + ch * CHUNK
        stp[b] = pltpu.async_copy(ob.at[pl.ds(b * CHUNK, CHUNK)],
                                  out_hbm.at[pl.ds(base, CHUNK)], st[b])
    for s in stp:
        if s is not None:
            s.wait()


@jax.jit
def kernel(x, Wg, W1, b1, W2, b2):
    x2 = x.reshape(N, D)
    wgt = Wg.reshape(TE, D).T
    pos0, pos1, w0v, w1v, gid64, aux = _routing(x2, wgt)
    pos0f = pos0.reshape(N)
    pos1f = pos1.reshape(N)
    gid = gid64[0, :NTILES]
    dispatch, combine = _sc_kernels()
    xs = dispatch(x2, pos0f, pos1f)
    ys = _grouped_mlp(gid, xs, W1, W2,
                      b1.reshape(E, 1, H), b2.reshape(E, 1, D))
    out2 = combine(ys, pos0f, pos1f, w0v, w1v)
    return out2.reshape(B, T, D), aux[0, 0]


# speedup vs baseline: 1.1449x; 1.0361x over previous
import functools

import jax
import jax.numpy as jnp
from jax import lax
from jax.experimental import pallas as pl
from jax.experimental.pallas import tpu as pltpu
from jax.experimental.pallas import tpu_sc as plsc

B, T, D, H, E, TOPK = 256, 8, 1024, 2048, 8, 2
N = B * T
TILE = 256
NT = N // TILE
TE = T * E
NEG = -1e30
BM = 128
A = N * TOPK
NTILES = A // BM + E
P = NTILES * BM
NW = 32
TPW = N // NW
CHUNK = 16
NCH = TPW // CHUNK


def _routing_body(x_ref, wgt_ref,
                  pos0_ref, pos1_ref, w0_ref, w1_ref, gid_ref, aux_ref,
                  e1s, e2s, r0s, r1s, w0s, w1s, rb, ps):
    i = pl.program_id(0)

    @pl.when(i == 0)
    def _():
        rb[...] = jnp.zeros((1, E), jnp.int32)
        ps[...] = jnp.zeros((1, TE), jnp.float32)

    @pl.when(i < NT)
    def _():
        xt = x_ref[...]
        logits = jnp.dot(xt, wgt_ref[...],
                         preferred_element_type=jnp.float32)
        rows = jax.lax.broadcasted_iota(jnp.int32, (TILE, TE), 0)
        cols = jax.lax.broadcasted_iota(jnp.int32, (TILE, TE), 1)
        valid = (cols // E) == (rows % T)
        ml = jnp.where(valid, logits, NEG)
        m1 = jnp.max(ml, axis=1, keepdims=True)
        i1 = jnp.min(jnp.where(ml >= m1, cols, TE), axis=1, keepdims=True)
        ml2 = jnp.where(cols == i1, NEG, ml)
        m2 = jnp.max(ml2, axis=1, keepdims=True)
        i2 = jnp.min(jnp.where(ml2 >= m2, cols, TE), axis=1, keepdims=True)
        w1 = 1.0 / (1.0 + jnp.exp(m2 - m1))
        w2 = 1.0 - w1
        e1 = i1 % E
        e2 = i2 % E
        e8 = jax.lax.broadcasted_iota(jnp.int32, (TILE, E), 1)
        oh1 = (e8 == e1)
        oh2 = (e8 == e2)
        mtot = oh1.astype(jnp.float32) + oh2.astype(jnp.float32)
        rr = jax.lax.broadcasted_iota(jnp.int32, (TILE, TILE), 0)
        cc = jax.lax.broadcasted_iota(jnp.int32, (TILE, TILE), 1)
        ltri = (rr > cc).astype(jnp.float32)
        cex = jnp.dot(ltri, mtot,
                      preferred_element_type=jnp.float32).astype(jnp.int32)
        rbb = jnp.broadcast_to(rb[...], (TILE, E))
        rank0 = jnp.sum(jnp.where(oh1, cex + rbb, 0), axis=1, keepdims=True)
        rank1 = jnp.sum(jnp.where(oh2, cex + rbb, 0), axis=1, keepdims=True)
        colsel = (jax.lax.broadcasted_iota(jnp.int32, (TILE, NT), 1) == i)
        e1s[...] = jnp.where(colsel, e1, e1s[...])
        e2s[...] = jnp.where(colsel, e2, e2s[...])
        r0s[...] = jnp.where(colsel, rank0, r0s[...])
        r1s[...] = jnp.where(colsel, rank1, r1s[...])
        rb[...] = rb[...] + jnp.sum(mtot, axis=0, keepdims=True
                                    ).astype(jnp.int32)
        p = jnp.exp(ml - m1)
        p = p / jnp.sum(p, axis=1, keepdims=True)
        ps[...] = ps[...] + jnp.sum(p, axis=0, keepdims=True)
        w0s[...] = jnp.where(colsel, w1, w0s[...])
        w1s[...] = jnp.where(colsel, w2, w1s[...])

    @pl.when(i == NT)
    def _():
        cnt = rb[...]
        padcnt = ((cnt + (BM - 1)) // BM) * BM
        po = []
        run = jnp.int32(0)
        for e in range(E):
            po.append(run)
            run = run + padcnt[0, e]
        sel0 = jnp.zeros((TILE, NT), jnp.int32)
        sel1 = jnp.zeros((TILE, NT), jnp.int32)
        for e in range(E):
            sel0 = sel0 + jnp.where(e1s[...] == e, po[e], 0)
            sel1 = sel1 + jnp.where(e2s[...] == e, po[e], 0)
        pos0_ref[...] = jnp.transpose(r0s[...] + sel0, (1, 0))
        pos1_ref[...] = jnp.transpose(r1s[...] + sel1, (1, 0))
        oh = (jax.lax.broadcasted_iota(jnp.int32, (N, NT), 0) // TILE
              == jax.lax.broadcasted_iota(jnp.int32, (N, NT), 1)
              ).astype(jnp.float32)
        ones16 = jnp.ones((NT, 16), jnp.float32)

        def _tok16(ws):
            m = jnp.concatenate([ws] * NT, axis=0) * oh
            return jnp.dot(m, ones16, preferred_element_type=jnp.float32,
                           precision=jax.lax.Precision.HIGHEST)

        w0_ref[...] = _tok16(w0s[...])
        w1_ref[...] = _tok16(w1s[...])
        tcol = jax.lax.broadcasted_iota(jnp.int32, (1, TE), 1) * BM
        g = jnp.zeros((1, TE), jnp.int32)
        for e in range(E):
            g = g + (tcol >= (po[e] + padcnt[0, e])).astype(jnp.int32)
        n_used = (po[E - 1] + padcnt[0, E - 1]) // BM
        lane = jax.lax.broadcasted_iota(jnp.int32, (1, TE), 1)
        gid_ref[...] = jnp.where(lane == NTILES, n_used,
                                 jnp.minimum(g, E - 1))
        j64 = jax.lax.broadcasted_iota(jnp.int32, (TE, E), 0)
        k8 = jax.lax.broadcasted_iota(jnp.int32, (TE, E), 1)
        fold = ((j64 % E) == k8).astype(jnp.float32)
        avg = jnp.dot(ps[...], fold, preferred_element_type=jnp.float32,
                      precision=jax.lax.Precision.HIGHEST) / float(N)
        aux_ref[...] = jnp.broadcast_to(float(E) * jnp.sum(avg * avg),
                                        (1, TE))


def _routing(x2, wgt):
    return pl.pallas_call(
        _routing_body,
        grid=(NT + 1,),
        in_specs=[
            pl.BlockSpec((TILE, D), lambda i: (jnp.minimum(i, NT - 1), 0)),
            pl.BlockSpec((D, TE), lambda i: (0, 0)),
        ],
        out_specs=[
            pl.BlockSpec((NT, TILE), lambda i: (0, 0)),
            pl.BlockSpec((NT, TILE), lambda i: (0, 0)),
            pl.BlockSpec((N, 16), lambda i: (0, 0)),
            pl.BlockSpec((N, 16), lambda i: (0, 0)),
            pl.BlockSpec((1, TE), lambda i: (0, 0)),
            pl.BlockSpec((1, TE), lambda i: (0, 0)),
        ],
        out_shape=[
            jax.ShapeDtypeStruct((NT, TILE), jnp.int32),
            jax.ShapeDtypeStruct((NT, TILE), jnp.int32),
            jax.ShapeDtypeStruct((N, 16), jnp.float32),
            jax.ShapeDtypeStruct((N, 16), jnp.float32),
            jax.ShapeDtypeStruct((1, TE), jnp.int32),
            jax.ShapeDtypeStruct((1, TE), jnp.float32),
        ],
        scratch_shapes=[
            pltpu.VMEM((TILE, NT), jnp.int32),
            pltpu.VMEM((TILE, NT), jnp.int32),
            pltpu.VMEM((TILE, NT), jnp.int32),
            pltpu.VMEM((TILE, NT), jnp.int32),
            pltpu.VMEM((TILE, NT), jnp.float32),
            pltpu.VMEM((TILE, NT), jnp.float32),
            pltpu.VMEM((1, E), jnp.int32),
            pltpu.VMEM((1, TE), jnp.float32),
        ],
        compiler_params=pltpu.CompilerParams(
            dimension_semantics=("arbitrary",)),
    )(x2, wgt)


NSPLIT = 1
HS = H // NSPLIT


def _mlp_body(gid_ref, xs_ref, *refs):
    w1s = refs[:NSPLIT]
    w2s = refs[NSPLIT:2 * NSPLIT]
    b1_ref = refs[2 * NSPLIT]
    b2_ref = refs[2 * NSPLIT + 1]
    ys_ref = refs[2 * NSPLIT + 2]

    @pl.when(pl.program_id(0) < gid_ref[NTILES])
    def _():
        x = xs_ref[...]
        acc = jnp.broadcast_to(b2_ref[0], (BM, D))
        for j in range(NSPLIT):
            h = jnp.maximum(
                jnp.dot(x, w1s[j][0], preferred_element_type=jnp.float32)
                + b1_ref[0, :, j * HS:(j + 1) * HS], 0.0)
            acc = acc + jnp.dot(h, w2s[j][0],
                                preferred_element_type=jnp.float32)
        ys_ref[...] = acc


def _grouped_mlp(gid, xs, W1, W2, b1r, b2r):
    w1_specs = [
        pl.BlockSpec((1, D, HS), lambda g, s, j=j: (s[g], 0, j))
        for j in range(NSPLIT)
    ]
    w2_specs = [
        pl.BlockSpec((1, HS, D), lambda g, s, j=j: (s[g], j, 0))
        for j in range(NSPLIT)
    ]
    grid_spec = pltpu.PrefetchScalarGridSpec(
        num_scalar_prefetch=1,
        grid=(NTILES,),
        in_specs=[
            pl.BlockSpec((BM, D), lambda g, s: (g, 0)),
            *w1_specs,
            *w2_specs,
            pl.BlockSpec((1, 1, H), lambda g, s: (s[g], 0, 0)),
            pl.BlockSpec((1, 1, D), lambda g, s: (s[g], 0, 0)),
        ],
        out_specs=pl.BlockSpec((BM, D), lambda g, s: (g, 0)),
    )
    return pl.pallas_call(
        _mlp_body,
        grid_spec=grid_spec,
        out_shape=jax.ShapeDtypeStruct((P, D), jnp.float32),
        compiler_params=pltpu.CompilerParams(
            dimension_semantics=("arbitrary",)),
    )(gid, xs, *([W1] * NSPLIT), *([W2] * NSPLIT), b1r, b2r)


@functools.lru_cache(maxsize=None)
def _sc_kernels():
    mesh = plsc.VectorSubcoreMesh(core_axis_name="c", subcore_axis_name="s")
    dispatch = pl.kernel(
        _dispatch_body,
        mesh=mesh,
        out_type=jax.ShapeDtypeStruct((P, D), jnp.float32),
        scratch_types=[
            pltpu.VMEM((TPW, D), jnp.float32),
            pltpu.VMEM((TPW,), jnp.int32),
            pltpu.VMEM((TPW,), jnp.int32),
            pltpu.SemaphoreType.DMA,
            pltpu.SemaphoreType.DMA,
        ],
    )
    combine = pl.kernel(
        _combine_body,
        mesh=mesh,
        out_type=jax.ShapeDtypeStruct((N, D), jnp.float32),
        scratch_types=[
            pltpu.VMEM((2 * CHUNK, D), jnp.float32),
            pltpu.VMEM((2 * CHUNK, D), jnp.float32),
            pltpu.VMEM((2 * CHUNK, D), jnp.float32),
            pltpu.VMEM((2, CHUNK), jnp.int32),
            pltpu.VMEM((2, CHUNK), jnp.int32),
            pltpu.VMEM((2, CHUNK, 16), jnp.float32),
            pltpu.VMEM((2, CHUNK, 16), jnp.float32),
            pltpu.SemaphoreType.DMA,
            pltpu.SemaphoreType.DMA,
            pltpu.SemaphoreType.DMA,
            pltpu.SemaphoreType.DMA,
            pltpu.SemaphoreType.DMA,
            pltpu.SemaphoreType.DMA,
        ],
    )
    return dispatch, combine


def _dispatch_body(x_hbm, p0_hbm, p1_hbm, xs_hbm, xbuf, i0, i1, sem, sem2):
    wid = lax.axis_index("s") * 2 + lax.axis_index("c")
    tb = wid * TPW
    cx = pltpu.async_copy(x_hbm.at[pl.ds(tb, TPW)], xbuf, sem)
    pltpu.sync_copy(p0_hbm.at[pl.ds(tb, TPW)], i0)
    pltpu.sync_copy(p1_hbm.at[pl.ds(tb, TPW)], i1)
    cx.wait()
    c0 = pltpu.async_copy(xbuf, xs_hbm.at[i0], sem)
    c1 = pltpu.async_copy(xbuf, xs_hbm.at[i1], sem2)
    c0.wait()
    c1.wait()


def _combine_body(ys_hbm, p0_hbm, p1_hbm, w0_hbm, w1_hbm, out_hbm,
                  y0, y1, ob, i0, i1, wb0, wb1,
                  sg0a, sg0b, sg1a, sg1b, sta, stb):
    wid = lax.axis_index("s") * 2 + lax.axis_index("c")
    sg0 = (sg0a, sg0b)
    sg1 = (sg1a, sg1b)
    st = (sta, stb)

    def issue(ch):
        b = ch % 2
        base = wid * TPW + ch * CHUNK
        rows = pl.ds(b * CHUNK, CHUNK)
        pltpu.sync_copy(p0_hbm.at[pl.ds(base, CHUNK)], i0.at[b])
        pltpu.sync_copy(p1_hbm.at[pl.ds(base, CHUNK)], i1.at[b])
        pltpu.sync_copy(w0_hbm.at[pl.ds(base, CHUNK)], wb0.at[b])
        pltpu.sync_copy(w1_hbm.at[pl.ds(base, CHUNK)], wb1.at[b])
        c0 = pltpu.async_copy(ys_hbm.at[i0.at[b]], y0.at[rows], sg0[b])
        c1 = pltpu.async_copy(ys_hbm.at[i1.at[b]], y1.at[rows], sg1[b])
        return c0, c1

    pend = issue(0)
    stp = [None, None]
    for ch in range(NCH):
        b = ch % 2
        cur = pend
        if ch + 1 < NCH:
            pend = issue(ch + 1)
        cur[0].wait()
        cur[1].wait()
        if stp[b] is not None:
            stp[b].wait()
        for r in range(CHUNK):
            row = b * CHUNK + r
            w0r = wb0[b, r]
            w1r = wb1[b, r]

            @plsc.parallel_loop(0, D // 16, 1, unroll=8)
            def _(c):
                sl = pl.ds(c * 16, 16)
                ob[row, sl] = w0r * y0[row, sl] + w1r * y1[row, sl]

        base = wid * TPW + ch * CHUNK
        stp[b] = pltpu.async_copy(ob.at[pl.ds(b * CHUNK, CHUNK)],
                                  out_hbm.at[pl.ds(base, CHUNK)], st[b])
    for s in stp:
        if s is not None:
            s.wait()


@jax.jit
def kernel(x, Wg, W1, b1, W2, b2):
    x2 = x.reshape(N, D)
    wgt = Wg.reshape(TE, D).T
    pos0, pos1, w0v, w1v, gid64, aux = _routing(x2, wgt)
    pos0f = pos0.reshape(N)
    pos1f = pos1.reshape(N)
    gid = gid64[0, :NTILES + 1]
    dispatch, combine = _sc_kernels()
    xs = dispatch(x2, pos0f, pos1f)
    ys = _grouped_mlp(gid, xs, W1, W2,
                      b1.reshape(E, 1, H), b2.reshape(E, 1, D))
    out2 = combine(ys, pos0f, pos1f, w0v, w1v)
    return out2.reshape(B, T, D), aux[0, 0]


# speedup vs baseline: 1.1690x; 1.0210x over previous
import functools

import jax
import jax.numpy as jnp
from jax import lax
from jax.experimental import pallas as pl
from jax.experimental.pallas import tpu as pltpu
from jax.experimental.pallas import tpu_sc as plsc

B, T, D, H, E, TOPK = 256, 8, 1024, 2048, 8, 2
N = B * T
TILE = 1024
NT = N // TILE
TE = T * E
NEG = -1e30
BM = 128
A = N * TOPK
NTILES = A // BM + E
P = NTILES * BM
NW = 32
TPW = N // NW
CHUNK = 16
NCH = TPW // CHUNK


def _routing_body(x_ref, wgt_ref,
                  pos0_ref, pos1_ref, w0_ref, w1_ref, gid_ref, aux_ref,
                  e1s, e2s, r0s, r1s, w0s, w1s, rb, ps):
    i = pl.program_id(0)

    @pl.when(i == 0)
    def _():
        rb[...] = jnp.zeros((1, E), jnp.int32)
        ps[...] = jnp.zeros((1, TE), jnp.float32)

    @pl.when(i < NT)
    def _():
        xt = x_ref[...]
        logits = jnp.dot(xt, wgt_ref[...],
                         preferred_element_type=jnp.float32)
        rows = jax.lax.broadcasted_iota(jnp.int32, (TILE, TE), 0)
        cols = jax.lax.broadcasted_iota(jnp.int32, (TILE, TE), 1)
        valid = (cols // E) == (rows % T)
        ml = jnp.where(valid, logits, NEG)
        m1 = jnp.max(ml, axis=1, keepdims=True)
        i1 = jnp.min(jnp.where(ml >= m1, cols, TE), axis=1, keepdims=True)
        ml2 = jnp.where(cols == i1, NEG, ml)
        m2 = jnp.max(ml2, axis=1, keepdims=True)
        i2 = jnp.min(jnp.where(ml2 >= m2, cols, TE), axis=1, keepdims=True)
        w1 = 1.0 / (1.0 + jnp.exp(m2 - m1))
        w2 = 1.0 - w1
        e1 = i1 % E
        e2 = i2 % E
        e8 = jax.lax.broadcasted_iota(jnp.int32, (TILE, E), 1)
        oh1 = (e8 == e1)
        oh2 = (e8 == e2)
        mtot = oh1.astype(jnp.float32) + oh2.astype(jnp.float32)
        rr = jax.lax.broadcasted_iota(jnp.int32, (TILE, TILE), 0)
        cc = jax.lax.broadcasted_iota(jnp.int32, (TILE, TILE), 1)
        ltri = (rr > cc).astype(jnp.float32)
        cex = jnp.dot(ltri, mtot,
                      preferred_element_type=jnp.float32).astype(jnp.int32)
        rbb = jnp.broadcast_to(rb[...], (TILE, E))
        rank0 = jnp.sum(jnp.where(oh1, cex + rbb, 0), axis=1, keepdims=True)
        rank1 = jnp.sum(jnp.where(oh2, cex + rbb, 0), axis=1, keepdims=True)
        colsel = (jax.lax.broadcasted_iota(jnp.int32, (TILE, NT), 1) == i)
        e1s[...] = jnp.where(colsel, e1, e1s[...])
        e2s[...] = jnp.where(colsel, e2, e2s[...])
        r0s[...] = jnp.where(colsel, rank0, r0s[...])
        r1s[...] = jnp.where(colsel, rank1, r1s[...])
        rb[...] = rb[...] + jnp.sum(mtot, axis=0, keepdims=True
                                    ).astype(jnp.int32)
        p = jnp.exp(ml - m1)
        p = p / jnp.sum(p, axis=1, keepdims=True)
        ps[...] = ps[...] + jnp.sum(p, axis=0, keepdims=True)
        w0s[...] = jnp.where(colsel, w1, w0s[...])
        w1s[...] = jnp.where(colsel, w2, w1s[...])

    @pl.when(i == NT)
    def _():
        cnt = rb[...]
        padcnt = ((cnt + (BM - 1)) // BM) * BM
        po = []
        run = jnp.int32(0)
        for e in range(E):
            po.append(run)
            run = run + padcnt[0, e]
        sel0 = jnp.zeros((TILE, NT), jnp.int32)
        sel1 = jnp.zeros((TILE, NT), jnp.int32)
        for e in range(E):
            sel0 = sel0 + jnp.where(e1s[...] == e, po[e], 0)
            sel1 = sel1 + jnp.where(e2s[...] == e, po[e], 0)
        pos0_ref[...] = jnp.transpose(r0s[...] + sel0, (1, 0))
        pos1_ref[...] = jnp.transpose(r1s[...] + sel1, (1, 0))
        oh = (jax.lax.broadcasted_iota(jnp.int32, (N, NT), 0) // TILE
              == jax.lax.broadcasted_iota(jnp.int32, (N, NT), 1)
              ).astype(jnp.float32)
        ones16 = jnp.ones((NT, 16), jnp.float32)

        def _tok16(ws):
            m = jnp.concatenate([ws] * NT, axis=0) * oh
            return jnp.dot(m, ones16, preferred_element_type=jnp.float32,
                           precision=jax.lax.Precision.HIGHEST)

        w0_ref[...] = _tok16(w0s[...])
        w1_ref[...] = _tok16(w1s[...])
        tcol = jax.lax.broadcasted_iota(jnp.int32, (1, TE), 1) * BM
        g = jnp.zeros((1, TE), jnp.int32)
        for e in range(E):
            g = g + (tcol >= (po[e] + padcnt[0, e])).astype(jnp.int32)
        n_used = (po[E - 1] + padcnt[0, E - 1]) // BM
        lane = jax.lax.broadcasted_iota(jnp.int32, (1, TE), 1)
        gid_ref[...] = jnp.where(lane == NTILES, n_used,
                                 jnp.minimum(g, E - 1))
        j64 = jax.lax.broadcasted_iota(jnp.int32, (TE, E), 0)
        k8 = jax.lax.broadcasted_iota(jnp.int32, (TE, E), 1)
        fold = ((j64 % E) == k8).astype(jnp.float32)
        avg = jnp.dot(ps[...], fold, preferred_element_type=jnp.float32,
                      precision=jax.lax.Precision.HIGHEST) / float(N)
        aux_ref[...] = jnp.broadcast_to(float(E) * jnp.sum(avg * avg),
                                        (1, TE))


def _routing(x2, wgt):
    return pl.pallas_call(
        _routing_body,
        grid=(NT + 1,),
        in_specs=[
            pl.BlockSpec((TILE, D), lambda i: (jnp.minimum(i, NT - 1), 0)),
            pl.BlockSpec((D, TE), lambda i: (0, 0)),
        ],
        out_specs=[
            pl.BlockSpec((NT, TILE), lambda i: (0, 0)),
            pl.BlockSpec((NT, TILE), lambda i: (0, 0)),
            pl.BlockSpec((N, 16), lambda i: (0, 0)),
            pl.BlockSpec((N, 16), lambda i: (0, 0)),
            pl.BlockSpec((1, TE), lambda i: (0, 0)),
            pl.BlockSpec((1, TE), lambda i: (0, 0)),
        ],
        out_shape=[
            jax.ShapeDtypeStruct((NT, TILE), jnp.int32),
            jax.ShapeDtypeStruct((NT, TILE), jnp.int32),
            jax.ShapeDtypeStruct((N, 16), jnp.float32),
            jax.ShapeDtypeStruct((N, 16), jnp.float32),
            jax.ShapeDtypeStruct((1, TE), jnp.int32),
            jax.ShapeDtypeStruct((1, TE), jnp.float32),
        ],
        scratch_shapes=[
            pltpu.VMEM((TILE, NT), jnp.int32),
            pltpu.VMEM((TILE, NT), jnp.int32),
            pltpu.VMEM((TILE, NT), jnp.int32),
            pltpu.VMEM((TILE, NT), jnp.int32),
            pltpu.VMEM((TILE, NT), jnp.float32),
            pltpu.VMEM((TILE, NT), jnp.float32),
            pltpu.VMEM((1, E), jnp.int32),
            pltpu.VMEM((1, TE), jnp.float32),
        ],
        compiler_params=pltpu.CompilerParams(
            dimension_semantics=("arbitrary",)),
    )(x2, wgt)


NSPLIT = 1
HS = H // NSPLIT


def _mlp_body(gid_ref, xs_ref, *refs):
    w1s = refs[:NSPLIT]
    w2s = refs[NSPLIT:2 * NSPLIT]
    b1_ref = refs[2 * NSPLIT]
    b2_ref = refs[2 * NSPLIT + 1]
    ys_ref = refs[2 * NSPLIT + 2]

    @pl.when(pl.program_id(0) < gid_ref[NTILES])
    def _():
        x = xs_ref[...]
        acc = jnp.broadcast_to(b2_ref[0], (BM, D))
        for j in range(NSPLIT):
            h = jnp.maximum(
                jnp.dot(x, w1s[j][0], preferred_element_type=jnp.float32)
                + b1_ref[0, :, j * HS:(j + 1) * HS], 0.0)
            acc = acc + jnp.dot(h, w2s[j][0],
                                preferred_element_type=jnp.float32)
        ys_ref[...] = acc


def _grouped_mlp(gid, xs, W1, W2, b1r, b2r):
    w1_specs = [
        pl.BlockSpec((1, D, HS), lambda g, s, j=j: (s[g], 0, j))
        for j in range(NSPLIT)
    ]
    w2_specs = [
        pl.BlockSpec((1, HS, D), lambda g, s, j=j: (s[g], j, 0))
        for j in range(NSPLIT)
    ]
    grid_spec = pltpu.PrefetchScalarGridSpec(
        num_scalar_prefetch=1,
        grid=(NTILES,),
        in_specs=[
            pl.BlockSpec((BM, D), lambda g, s: (g, 0)),
            *w1_specs,
            *w2_specs,
            pl.BlockSpec((1, 1, H), lambda g, s: (s[g], 0, 0)),
            pl.BlockSpec((1, 1, D), lambda g, s: (s[g], 0, 0)),
        ],
        out_specs=pl.BlockSpec((BM, D), lambda g, s: (g, 0)),
    )
    return pl.pallas_call(
        _mlp_body,
        grid_spec=grid_spec,
        out_shape=jax.ShapeDtypeStruct((P, D), jnp.float32),
        compiler_params=pltpu.CompilerParams(
            dimension_semantics=("arbitrary",)),
    )(gid, xs, *([W1] * NSPLIT), *([W2] * NSPLIT), b1r, b2r)


@functools.lru_cache(maxsize=None)
def _sc_kernels():
    mesh = plsc.VectorSubcoreMesh(core_axis_name="c", subcore_axis_name="s")
    dispatch = pl.kernel(
        _dispatch_body,
        mesh=mesh,
        out_type=jax.ShapeDtypeStruct((P, D), jnp.float32),
        scratch_types=[
            pltpu.VMEM((TPW, D), jnp.float32),
            pltpu.VMEM((TPW,), jnp.int32),
            pltpu.VMEM((TPW,), jnp.int32),
            pltpu.SemaphoreType.DMA,
            pltpu.SemaphoreType.DMA,
        ],
    )
    combine = pl.kernel(
        _combine_body,
        mesh=mesh,
        out_type=jax.ShapeDtypeStruct((N, D), jnp.float32),
        scratch_types=[
            pltpu.VMEM((2 * CHUNK, D), jnp.float32),
            pltpu.VMEM((2 * CHUNK, D), jnp.float32),
            pltpu.VMEM((2 * CHUNK, D), jnp.float32),
            pltpu.VMEM((2, CHUNK), jnp.int32),
            pltpu.VMEM((2, CHUNK), jnp.int32),
            pltpu.VMEM((2, CHUNK, 16), jnp.float32),
            pltpu.VMEM((2, CHUNK, 16), jnp.float32),
            pltpu.SemaphoreType.DMA,
            pltpu.SemaphoreType.DMA,
            pltpu.SemaphoreType.DMA,
            pltpu.SemaphoreType.DMA,
            pltpu.SemaphoreType.DMA,
            pltpu.SemaphoreType.DMA,
        ],
    )
    return dispatch, combine


def _dispatch_body(x_hbm, p0_hbm, p1_hbm, xs_hbm, xbuf, i0, i1, sem, sem2):
    wid = lax.axis_index("s") * 2 + lax.axis_index("c")
    tb = wid * TPW
    cx = pltpu.async_copy(x_hbm.at[pl.ds(tb, TPW)], xbuf, sem)
    pltpu.sync_copy(p0_hbm.at[pl.ds(tb, TPW)], i0)
    pltpu.sync_copy(p1_hbm.at[pl.ds(tb, TPW)], i1)
    cx.wait()
    c0 = pltpu.async_copy(xbuf, xs_hbm.at[i0], sem)
    c1 = pltpu.async_copy(xbuf, xs_hbm.at[i1], sem2)
    c0.wait()
    c1.wait()


def _combine_body(ys_hbm, p0_hbm, p1_hbm, w0_hbm, w1_hbm, out_hbm,
                  y0, y1, ob, i0, i1, wb0, wb1,
                  sg0a, sg0b, sg1a, sg1b, sta, stb):
    wid = lax.axis_index("s") * 2 + lax.axis_index("c")
    sg0 = (sg0a, sg0b)
    sg1 = (sg1a, sg1b)
    st = (sta, stb)

    def issue(ch):
        b = ch % 2
        base = wid * TPW + ch * CHUNK
        rows = pl.ds(b * CHUNK, CHUNK)
        pltpu.sync_copy(p0_hbm.at[pl.ds(base, CHUNK)], i0.at[b])
        pltpu.sync_copy(p1_hbm.at[pl.ds(base, CHUNK)], i1.at[b])
        pltpu.sync_copy(w0_hbm.at[pl.ds(base, CHUNK)], wb0.at[b])
        pltpu.sync_copy(w1_hbm.at[pl.ds(base, CHUNK)], wb1.at[b])
        c0 = pltpu.async_copy(ys_hbm.at[i0.at[b]], y0.at[rows], sg0[b])
        c1 = pltpu.async_copy(ys_hbm.at[i1.at[b]], y1.at[rows], sg1[b])
        return c0, c1

    pend = issue(0)
    stp = [None, None]
    for ch in range(NCH):
        b = ch % 2
        cur = pend
        if ch + 1 < NCH:
            pend = issue(ch + 1)
        cur[0].wait()
        cur[1].wait()
        if stp[b] is not None:
            stp[b].wait()
        for r in range(CHUNK):
            row = b * CHUNK + r
            w0r = wb0[b, r]
            w1r = wb1[b, r]

            @plsc.parallel_loop(0, D // 16, 1, unroll=8)
            def _(c):
                sl = pl.ds(c * 16, 16)
                ob[row, sl] = w0r * y0[row, sl] + w1r * y1[row, sl]

        base = wid * TPW + ch * CHUNK
        stp[b] = pltpu.async_copy(ob.at[pl.ds(b * CHUNK, CHUNK)],
                                  out_hbm.at[pl.ds(base, CHUNK)], st[b])
    for s in stp:
        if s is not None:
            s.wait()


@jax.jit
def kernel(x, Wg, W1, b1, W2, b2):
    x2 = x.reshape(N, D)
    wgt = Wg.reshape(TE, D).T
    pos0, pos1, w0v, w1v, gid64, aux = _routing(x2, wgt)
    pos0f = pos0.reshape(N)
    pos1f = pos1.reshape(N)
    gid = gid64[0, :NTILES + 1]
    dispatch, combine = _sc_kernels()
    xs = dispatch(x2, pos0f, pos1f)
    ys = _grouped_mlp(gid, xs, W1, W2,
                      b1.reshape(E, 1, H), b2.reshape(E, 1, D))
    out2 = combine(ys, pos0f, pos1f, w0v, w1v)
    return out2.reshape(B, T, D), aux[0, 0]
